# Initial kernel scaffold; baseline (speedup 1.0000x reference)
import jax
import jax.numpy as jnp
from jax.experimental import pallas as pl


def kernel(log_weights, ancestral_indices):
    raise NotImplementedError("write your pallas kernel here")


# speedup vs baseline: 2.7224x; 2.7224x over previous
import functools

import jax
import jax.numpy as jnp
from jax import lax
from jax.experimental import pallas as pl
from jax.experimental.pallas import tpu as pltpu
from jax.experimental.pallas import tpu_sc as plsc

T = 50
B = 128
K = 2048
NC = 2
NS = 16
NW = NC * NS
BPW = B // NW
LANES = 16
CHUNKS = K // LANES


def _gather_sc(lw, idx):
    mesh = plsc.VectorSubcoreMesh(core_axis_name="c", subcore_axis_name="s")

    @functools.partial(
        pl.kernel,
        out_type=jax.ShapeDtypeStruct((NW, LANES), jnp.float32),
        mesh=mesh,
        scratch_types=[
            pltpu.VMEM((BPW * K,), jnp.float32),
            pltpu.VMEM((BPW * K,), jnp.int32),
            pltpu.VMEM((LANES,), jnp.float32),
            pltpu.SemaphoreType.DMA,
        ],
        compiler_params=pltpu.CompilerParams(needs_layout_passes=False),
    )
    def body(lw_hbm, idx_hbm, out_hbm, lw_v, idx_v, out_v, sem):
        wid = lax.axis_index("s") * NC + lax.axis_index("c")
        b0 = wid * BPW

        lane = lax.iota(jnp.int32, LANES)

        def t_body(t, accs):
            pltpu.sync_copy(lw_hbm.at[t, pl.ds(b0 * K, BPW * K)], lw_v)
            pltpu.sync_copy(idx_hbm.at[t, pl.ds(b0 * K, BPW * K)], idx_v)
            new_accs = []
            for j in range(BPW):

                def chunk_body(i, acc, j=j):
                    off = pl.multiple_of(j * K + i * LANES, LANES)
                    iv = idx_v[pl.ds(off, LANES)] + (j * K)
                    vals = plsc.load_gather(lw_v, [iv])
                    return acc + vals

                new_accs.append(
                    lax.fori_loop(0, CHUNKS, chunk_body, accs[j], unroll=8)
                )
            return tuple(new_accs)

        zero = jnp.zeros((LANES,), jnp.float32)
        accs = lax.fori_loop(0, T - 1, t_body, (zero,) * BPW)

        out_vec = jnp.zeros((LANES,), jnp.float32)
        for j in range(BPW):
            out_vec = jnp.where(lane == j, jnp.sum(accs[j]), out_vec)
        out_v[...] = out_vec
        pltpu.sync_copy(out_v, out_hbm.at[wid])

    return body(lw, idx)


def _lse_tc(lw):
    def body(lw_ref, out_ref):
        t = pl.program_id(0)
        x = lw_ref[0]
        m = jnp.max(x, axis=1, keepdims=True)
        s = jnp.sum(jnp.exp(x - m), axis=1)
        lse = m[:, 0] + jnp.log(s)

        @pl.when(t == 0)
        def _():
            out_ref[...] = jnp.zeros_like(out_ref)

        out_ref[0, :] += float(K) * lse

    return pl.pallas_call(
        body,
        grid=(T - 1,),
        in_specs=[pl.BlockSpec((1, B, K), lambda t: (t, 0, 0))],
        out_specs=pl.BlockSpec((1, B), lambda t: (0, 0)),
        out_shape=jax.ShapeDtypeStruct((1, B), jnp.float32),
    )(lw)


def kernel(log_weights, ancestral_indices):
    gat = _gather_sc(log_weights.reshape(T, B * K),
                     ancestral_indices.reshape(T - 1, B * K))
    lse = _lse_tc(log_weights)
    return gat[:, :BPW].reshape(B) - lse[0]


# speedup vs baseline: 10.4786x; 3.8490x over previous
import functools

import jax
import jax.numpy as jnp
from jax import lax
from jax.experimental import pallas as pl
from jax.experimental.pallas import tpu as pltpu
from jax.experimental.pallas import tpu_sc as plsc

T = 50
B = 128
K = 2048
NC = 2
NS = 16
NW = NC * NS
BPW = B // NW
LANES = 16
CHUNKS = K // LANES


def _gather_sc(lw, idx):
    mesh = plsc.VectorSubcoreMesh(core_axis_name="c", subcore_axis_name="s")

    @functools.partial(
        pl.kernel,
        out_type=jax.ShapeDtypeStruct((NW, LANES), jnp.float32),
        mesh=mesh,
        scratch_types=[
            pltpu.VMEM((BPW * K,), jnp.float32),
            pltpu.VMEM((BPW * K,), jnp.float32),
            pltpu.VMEM((BPW * K,), jnp.int32),
            pltpu.VMEM((BPW * K,), jnp.int32),
            pltpu.VMEM((LANES,), jnp.float32),
            pltpu.SemaphoreType.DMA,
            pltpu.SemaphoreType.DMA,
        ],
        compiler_params=pltpu.CompilerParams(needs_layout_passes=False),
    )
    def body(lw_hbm, idx_hbm, out_hbm, lw0, lw1, idx0, idx1, out_v, sem0, sem1):
        wid = lax.axis_index("s") * NC + lax.axis_index("c")
        b0 = wid * BPW
        lane = lax.iota(jnp.int32, LANES)
        lw_bufs, idx_bufs, sems = (lw0, lw1), (idx0, idx1), (sem0, sem1)

        def issue(t, phase):
            for j in range(BPW):
                dst = pl.ds(j * K, K)
                pltpu.async_copy(lw_hbm.at[t, b0 + j], lw_bufs[phase].at[dst],
                                 sems[phase])
                pltpu.async_copy(idx_hbm.at[t, b0 + j], idx_bufs[phase].at[dst],
                                 sems[phase])

        def drain(t, phase):
            for j in range(BPW):
                dst = pl.ds(j * K, K)
                pltpu.make_async_copy(lw_hbm.at[t, b0 + j],
                                      lw_bufs[phase].at[dst], sems[phase]).wait()
                pltpu.make_async_copy(idx_hbm.at[t, b0 + j],
                                      idx_bufs[phase].at[dst], sems[phase]).wait()

        def compute(phase, accs):
            lw_v, idx_v = lw_bufs[phase], idx_bufs[phase]
            new_accs = []
            for j in range(BPW):

                def chunk_body(i, acc, j=j):
                    off = pl.multiple_of(j * K + i * LANES, LANES)
                    iv = idx_v[pl.ds(off, LANES)] + (j * K)
                    vals = plsc.load_gather(lw_v, [iv])
                    return acc + vals

                new_accs.append(
                    lax.fori_loop(0, CHUNKS, chunk_body, accs[j], unroll=8)
                )
            return tuple(new_accs)

        issue(0, 0)
        issue(1, 1)

        def pair_body(tp, accs):
            t = 2 * tp
            drain(t, 0)
            accs = compute(0, accs)
            issue(t + 2, 0)
            drain(t + 1, 1)
            accs = compute(1, accs)
            issue(jnp.minimum(t + 3, T - 2), 1)
            return accs

        zero = jnp.zeros((LANES,), jnp.float32)
        accs = lax.fori_loop(0, (T - 1) // 2, pair_body, (zero,) * BPW)
        drain(T - 2, 0)
        accs = compute(0, accs)
        drain(T - 2, 1)

        out_vec = jnp.zeros((LANES,), jnp.float32)
        for j in range(BPW):
            out_vec = jnp.where(lane == j, jnp.sum(accs[j]), out_vec)
        out_v[...] = out_vec
        pltpu.sync_copy(out_v, out_hbm.at[wid])

    return body(lw, idx)


def _lse_tc(lw):
    def body(lw_ref, out_ref):
        t = pl.program_id(0)
        x = lw_ref[0]
        m = jnp.max(x, axis=1, keepdims=True)
        s = jnp.sum(jnp.exp(x - m), axis=1)
        lse = m[:, 0] + jnp.log(s)

        @pl.when(t == 0)
        def _():
            out_ref[...] = jnp.zeros_like(out_ref)

        out_ref[0, :] += float(K) * lse

    return pl.pallas_call(
        body,
        grid=(T - 1,),
        in_specs=[pl.BlockSpec((1, B, K), lambda t: (t, 0, 0))],
        out_specs=pl.BlockSpec((1, B), lambda t: (0, 0)),
        out_shape=jax.ShapeDtypeStruct((1, B), jnp.float32),
    )(lw)


def kernel(log_weights, ancestral_indices):
    gat = _gather_sc(log_weights, ancestral_indices)
    lse = _lse_tc(log_weights)
    return gat[:, :BPW].reshape(B) - lse[0]
